# Initial kernel scaffold; baseline (speedup 1.0000x reference)
import jax
import jax.numpy as jnp
from jax.experimental import pallas as pl


def kernel(x, weight, bias):
    raise NotImplementedError("write your pallas kernel here")


# speedup vs baseline: 12.8129x; 12.8129x over previous
import jax
import jax.numpy as jnp
from jax.experimental import pallas as pl

_C1 = 1024
_C2 = 1024
_K = 128
_TL = 256


def _body(x_ref, w_ref, b_ref, o_ref):
    x = x_ref[...]
    a = jax.lax.bitcast_convert_type(x, jnp.int32) & jnp.int32(0x7FFFFFFF)

    def search_bits(i, carry):
        lo, hi = carry
        v = lo + (hi - lo + 1) // 2
        cnt = jnp.sum((a >= v).astype(jnp.int32), axis=1, keepdims=True)
        ge = cnt >= _K
        return jnp.where(ge, v, lo), jnp.where(ge, hi, v - 1)

    lo0 = jnp.zeros((_TL, 1), jnp.int32)
    hi0 = jnp.full((_TL, 1), 0x7F800000, jnp.int32)
    t, _ = jax.lax.fori_loop(0, 31, search_bits, (lo0, hi0))

    n_gt = jnp.sum((a > t).astype(jnp.int32), axis=1, keepdims=True)
    needed = _K - n_gt
    idx = jax.lax.broadcasted_iota(jnp.int32, (_TL, _C1), 1)
    eq = a == t

    def search_idx(i, carry):
        lo, hi = carry
        m = (lo + hi) // 2
        cnt = jnp.sum((eq & (idx <= m)).astype(jnp.int32), axis=1, keepdims=True)
        ok = cnt >= needed
        return jnp.where(ok, lo, m + 1), jnp.where(ok, m, hi)

    lo1 = jnp.zeros((_TL, 1), jnp.int32)
    hi1 = jnp.full((_TL, 1), _C1 - 1, jnp.int32)
    m, _ = jax.lax.fori_loop(0, 10, search_idx, (lo1, hi1))

    mask = (a > t) | (eq & (idx <= m))
    xm = jnp.where(mask, x, 0.0)
    o_ref[...] = (
        jnp.dot(xm, w_ref[...], preferred_element_type=jnp.float32) + b_ref[...]
    )


def kernel(x, weight, bias):
    b, l, c1 = x.shape
    x2 = x.reshape(b * l, c1)
    out = pl.pallas_call(
        _body,
        grid=((b * l) // _TL,),
        in_specs=[
            pl.BlockSpec((_TL, _C1), lambda i: (i, 0)),
            pl.BlockSpec((_C1, _C2), lambda i: (0, 0)),
            pl.BlockSpec((1, _C2), lambda i: (0, 0)),
        ],
        out_specs=pl.BlockSpec((_TL, _C2), lambda i: (i, 0)),
        out_shape=jax.ShapeDtypeStruct((b * l, _C2), jnp.float32),
    )(x2, weight, bias.reshape(1, _C2))
    return out.reshape(b, l, _C2)


# speedup vs baseline: 17.4048x; 1.3584x over previous
import jax
import jax.numpy as jnp
from jax.experimental import pallas as pl

_C1 = 1024
_C2 = 1024
_K = 128
_TL = 256


def _body(x_ref, w_ref, b_ref, o_ref):
    x = x_ref[...]
    xt = x.T
    at = jax.lax.bitcast_convert_type(xt, jnp.int32) & jnp.int32(0x7FFFFFFF)
    ones = jnp.ones((8, _C1), jnp.bfloat16)

    def count_ge(mask):
        mbf = mask.astype(jnp.bfloat16)
        c = jax.lax.dot_general(
            ones, mbf, (((1,), (0,)), ((), ())),
            preferred_element_type=jnp.float32,
        )
        return c[0:1, :]

    kf = jnp.float32(_K)

    def search_bits(i, carry):
        lo, hi = carry
        v = lo + (hi - lo + 1) // 2
        ge = count_ge(at >= v) >= kf
        return jnp.where(ge, v, lo), jnp.where(ge, hi, v - 1)

    lo0 = jnp.zeros((1, _TL), jnp.int32)
    hi0 = jnp.full((1, _TL), 0x7F800000, jnp.int32)
    t, _ = jax.lax.fori_loop(0, 31, search_bits, (lo0, hi0))

    gt = at > t
    eq = at == t
    needed = kf - count_ge(gt)
    idx = jax.lax.broadcasted_iota(jnp.int32, (_C1, _TL), 0)

    def search_idx(i, carry):
        lo, hi = carry
        m = (lo + hi) // 2
        ok = count_ge(eq & (idx <= m)) >= needed
        return jnp.where(ok, lo, m + 1), jnp.where(ok, m, hi)

    lo1 = jnp.zeros((1, _TL), jnp.int32)
    hi1 = jnp.full((1, _TL), _C1 - 1, jnp.int32)
    m, _ = jax.lax.fori_loop(0, 10, search_idx, (lo1, hi1))

    mask = gt | (eq & (idx <= m))
    xmt = jnp.where(mask, xt, 0.0)
    out = jax.lax.dot_general(
        xmt, w_ref[...], (((0,), (0,)), ((), ())),
        preferred_element_type=jnp.float32,
    )
    o_ref[...] = out + b_ref[...]


def kernel(x, weight, bias):
    b, l, c1 = x.shape
    x2 = x.reshape(b * l, c1)
    out = pl.pallas_call(
        _body,
        grid=((b * l) // _TL,),
        in_specs=[
            pl.BlockSpec((_TL, _C1), lambda i: (i, 0)),
            pl.BlockSpec((_C1, _C2), lambda i: (0, 0)),
            pl.BlockSpec((1, _C2), lambda i: (0, 0)),
        ],
        out_specs=pl.BlockSpec((_TL, _C2), lambda i: (i, 0)),
        out_shape=jax.ShapeDtypeStruct((b * l, _C2), jnp.float32),
    )(x2, weight, bias.reshape(1, _C2))
    return out.reshape(b, l, _C2)


# speedup vs baseline: 22.1981x; 1.2754x over previous
import jax
import jax.numpy as jnp
from jax.experimental import pallas as pl

_C1 = 1024
_C2 = 1024
_K = 128
_TL = 512


def _body(x_ref, w_ref, b_ref, o_ref):
    x = x_ref[...]
    xt = x.T
    at = jax.lax.bitcast_convert_type(xt, jnp.int32) & jnp.int32(0x7FFFFFFF)
    ones = jnp.ones((8, _C1), jnp.bfloat16)

    def count_ge(mask):
        mbf = mask.astype(jnp.bfloat16)
        c = jax.lax.dot_general(
            ones, mbf, (((1,), (0,)), ((), ())),
            preferred_element_type=jnp.float32,
        )
        return c[0:1, :]

    kf = jnp.float32(_K)

    def search_bits(i, carry):
        lo, hi = carry
        v = lo + (hi - lo + 1) // 2
        ge = count_ge(at >= v) >= kf
        return jnp.where(ge, v, lo), jnp.where(ge, hi, v - 1)

    lo0 = jnp.zeros((1, _TL), jnp.int32)
    hi0 = jnp.full((1, _TL), 0x7F800000, jnp.int32)
    t, _ = jax.lax.fori_loop(0, 31, search_bits, (lo0, hi0))

    gt = at > t
    eq = at == t
    needed = kf - count_ge(gt)
    idx = jax.lax.broadcasted_iota(jnp.int32, (_C1, _TL), 0)

    def search_idx(i, carry):
        lo, hi = carry
        m = (lo + hi) // 2
        ok = count_ge(eq & (idx <= m)) >= needed
        return jnp.where(ok, lo, m + 1), jnp.where(ok, m, hi)

    lo1 = jnp.zeros((1, _TL), jnp.int32)
    hi1 = jnp.full((1, _TL), _C1 - 1, jnp.int32)
    m, _ = jax.lax.fori_loop(0, 10, search_idx, (lo1, hi1))

    mask = gt | (eq & (idx <= m))
    xmt = jnp.where(mask, xt, 0.0)
    out = jax.lax.dot_general(
        xmt, w_ref[...], (((0,), (0,)), ((), ())),
        preferred_element_type=jnp.float32,
    )
    o_ref[...] = out + b_ref[...]


def kernel(x, weight, bias):
    b, l, c1 = x.shape
    x2 = x.reshape(b * l, c1)
    out = pl.pallas_call(
        _body,
        grid=((b * l) // _TL,),
        in_specs=[
            pl.BlockSpec((_TL, _C1), lambda i: (i, 0)),
            pl.BlockSpec((_C1, _C2), lambda i: (0, 0)),
            pl.BlockSpec((1, _C2), lambda i: (0, 0)),
        ],
        out_specs=pl.BlockSpec((_TL, _C2), lambda i: (i, 0)),
        out_shape=jax.ShapeDtypeStruct((b * l, _C2), jnp.float32),
    )(x2, weight, bias.reshape(1, _C2))
    return out.reshape(b, l, _C2)


# speedup vs baseline: 27.8558x; 1.2549x over previous
import jax
import jax.numpy as jnp
from jax.experimental import pallas as pl

_C1 = 1024
_C2 = 1024
_K = 128
_TL = 512


def _body(x_ref, w_ref, b_ref, tri_ref, o_ref):
    x = x_ref[...]
    xt = x.T
    at = jax.lax.bitcast_convert_type(xt, jnp.int32) & jnp.int32(0x7FFFFFFF)
    ones = jnp.ones((8, _C1), jnp.bfloat16)
    kf = jnp.float32(_K)

    def count(mask):
        mbf = mask.astype(jnp.bfloat16)
        c = jax.lax.dot_general(
            ones, mbf, (((1,), (0,)), ((), ())),
            preferred_element_type=jnp.float32,
        )
        return c[0:1, :]

    hi_bits = (at >> 16).astype(jnp.int16)

    def search_hi(b, t):
        cand = t | (1 << (14 - b))
        ge = count(hi_bits >= cand.astype(jnp.int16)) >= kf
        return jnp.where(ge, cand, t)

    t_hi32 = jax.lax.fori_loop(0, 15, search_hi, jnp.zeros((1, _TL), jnp.int32))
    t_hi = t_hi32.astype(jnp.int16)

    eq_hi = hi_bits == t_hi
    gt_hi = hi_bits > t_hi
    n_gt_hi = count(gt_hi)
    kf2 = kf - n_gt_hi
    lo_bits = ((at & 0xFFFF) - 32768).astype(jnp.int16)
    key16 = jnp.where(eq_hi, lo_bits, jnp.int16(-32768))

    def search_lo(b, tu):
        cand = tu | (1 << (15 - b))
        cand_s = (cand - 32768).astype(jnp.int16)
        ge = count(key16 >= cand_s) >= kf2
        return jnp.where(ge, cand, tu)

    tu = jax.lax.fori_loop(0, 16, search_lo, jnp.zeros((1, _TL), jnp.int32))
    t_lo = (tu - 32768).astype(jnp.int16)

    gt = gt_hi | (key16 > t_lo)
    eq = eq_hi & (key16 == t_lo)
    needed = kf - count(gt)
    eqb = eq.astype(jnp.bfloat16)
    prefix = jax.lax.dot_general(
        tri_ref[...], eqb, (((1,), (0,)), ((), ())),
        preferred_element_type=jnp.float32,
    )
    mask = gt | (eq & (prefix <= needed))
    xmt = jnp.where(mask, xt, 0.0)
    out = jax.lax.dot_general(
        xmt, w_ref[...], (((0,), (0,)), ((), ())),
        preferred_element_type=jnp.float32,
    )
    o_ref[...] = out + b_ref[...]


def kernel(x, weight, bias):
    b, l, c1 = x.shape
    x2 = x.reshape(b * l, c1)
    r = jax.lax.broadcasted_iota(jnp.int32, (_C1, _C1), 0)
    c = jax.lax.broadcasted_iota(jnp.int32, (_C1, _C1), 1)
    tri = (r >= c).astype(jnp.bfloat16)
    out = pl.pallas_call(
        _body,
        grid=((b * l) // _TL,),
        in_specs=[
            pl.BlockSpec((_TL, _C1), lambda i: (i, 0)),
            pl.BlockSpec((_C1, _C2), lambda i: (0, 0)),
            pl.BlockSpec((1, _C2), lambda i: (0, 0)),
            pl.BlockSpec((_C1, _C1), lambda i: (0, 0)),
        ],
        out_specs=pl.BlockSpec((_TL, _C2), lambda i: (i, 0)),
        out_shape=jax.ShapeDtypeStruct((b * l, _C2), jnp.float32),
    )(x2, weight, bias.reshape(1, _C2), tri)
    return out.reshape(b, l, _C2)
